# Initial kernel scaffold; baseline (speedup 1.0000x reference)
import jax
import jax.numpy as jnp
from jax.experimental import pallas as pl


def kernel(h, t, r, ent_weight, rel_weight, norm_weight):
    raise NotImplementedError("write your pallas kernel here")


# speedup vs baseline: 2.7730x; 2.7730x over previous
import jax
import jax.numpy as jnp
from jax import lax
from jax.experimental import pallas as pl
from jax.experimental.pallas import tpu as pltpu
from jax.experimental.pallas import tpu_sc as plsc

EMB_DIM = 128
BATCH_N = 16384
N_CORES = 2
N_SUBCORES = 16
LANES = 16
SEGS = EMB_DIM // LANES
N_WORKERS = N_CORES * N_SUBCORES
ROWS_PER_WORKER = BATCH_N // N_WORKERS
CHUNK = 64
N_CHUNKS = ROWS_PER_WORKER // CHUNK
GROUPS = CHUNK // LANES


def _allsum(x, lanes):
    del lanes
    return jnp.sum(x)


def _sc_body(h_hbm, t_hbm, r_hbm, ent_hbm, reln_hbm, out_hbm,
             hi_v, ti_v, ri_v, hrows, trows, rnrows, score_v, sem):
    wid = lax.axis_index("s") * N_CORES + lax.axis_index("c")
    base = wid * ROWS_PER_WORKER
    lanes = lax.iota(jnp.int32, LANES)

    def chunk_body(ck, carry):
        off = base + ck * CHUNK
        pltpu.sync_copy(h_hbm.at[pl.ds(off, CHUNK)], hi_v)
        pltpu.sync_copy(t_hbm.at[pl.ds(off, CHUNK)], ti_v)
        pltpu.sync_copy(r_hbm.at[pl.ds(off, CHUNK)], ri_v)
        cp1 = pltpu.async_copy(ent_hbm.at[hi_v], hrows, sem)
        cp2 = pltpu.async_copy(ent_hbm.at[ti_v], trows, sem)
        cp3 = pltpu.async_copy(reln_hbm.at[ri_v], rnrows, sem)
        cp1.wait()
        cp2.wait()
        cp3.wait()

        def group_body(g, carry2):
            gbase = g * LANES
            score_vec = jnp.zeros((LANES,), jnp.float32)
            for k in range(LANES):
                i = gbase + k
                d = [hrows[i, pl.ds(j * LANES, LANES)]
                     - trows[i, pl.ds(j * LANES, LANES)]
                     for j in range(SEGS)]
                n = [rnrows[i, pl.ds(EMB_DIM + j * LANES, LANES)]
                     for j in range(SEGS)]
                dot = d[0] * n[0]
                for j in range(1, SEGS):
                    dot = dot + d[j] * n[j]
                s = _allsum(dot, lanes)
                acc = jnp.zeros((LANES,), jnp.float32)
                for j in range(SEGS):
                    rv = rnrows[i, pl.ds(j * LANES, LANES)]
                    acc = acc + jnp.abs(d[j] + rv - s * n[j])
                sc = _allsum(acc, lanes)
                score_vec = jnp.where(lanes == k, sc, score_vec)
            score_v[pl.ds(gbase, LANES)] = score_vec
            return carry2

        lax.fori_loop(0, GROUPS, group_body, 0)
        pltpu.sync_copy(score_v, out_hbm.at[pl.ds(off, CHUNK)])
        return carry

    lax.fori_loop(0, N_CHUNKS, chunk_body, 0)


def kernel(h, t, r, ent_weight, rel_weight, norm_weight):
    reln = jnp.concatenate([rel_weight, norm_weight], axis=1)
    mesh = plsc.VectorSubcoreMesh(core_axis_name="c", subcore_axis_name="s")
    run = pl.kernel(
        _sc_body,
        out_type=jax.ShapeDtypeStruct((BATCH_N,), jnp.float32),
        mesh=mesh,
        compiler_params=pltpu.CompilerParams(needs_layout_passes=False),
        scratch_types=[
            pltpu.VMEM((CHUNK,), jnp.int32),
            pltpu.VMEM((CHUNK,), jnp.int32),
            pltpu.VMEM((CHUNK,), jnp.int32),
            pltpu.VMEM((CHUNK, EMB_DIM), jnp.float32),
            pltpu.VMEM((CHUNK, EMB_DIM), jnp.float32),
            pltpu.VMEM((CHUNK, 2 * EMB_DIM), jnp.float32),
            pltpu.VMEM((CHUNK,), jnp.float32),
            pltpu.SemaphoreType.DMA,
        ],
    )
    return run(h.astype(jnp.int32), t.astype(jnp.int32),
               r.astype(jnp.int32), ent_weight, reln)


# speedup vs baseline: 3.6797x; 1.3270x over previous
import jax
import jax.numpy as jnp
from jax import lax
from jax.experimental import pallas as pl
from jax.experimental.pallas import tpu as pltpu
from jax.experimental.pallas import tpu_sc as plsc

EMB_DIM = 128
BATCH_N = 16384
N_CORES = 2
N_SUBCORES = 16
LANES = 16
SEGS = EMB_DIM // LANES
N_WORKERS = N_CORES * N_SUBCORES
ROWS_PER_WORKER = BATCH_N // N_WORKERS
CHUNK = 64
N_CHUNKS = ROWS_PER_WORKER // CHUNK
GROUPS = CHUNK // LANES


def _sc_body(h3, t3, r3, ent_hbm, reln_hbm, out_hbm,
             hi_all, ti_all, ri_all,
             hr0, tr0, rn0, hr1, tr1, rn1, score_all, sem0, sem1):
    wid = lax.axis_index("s") * N_CORES + lax.axis_index("c")
    base = wid * ROWS_PER_WORKER
    lanes = lax.iota(jnp.int32, LANES)

    pltpu.sync_copy(h3.at[wid], hi_all)
    pltpu.sync_copy(t3.at[wid], ti_all)
    pltpu.sync_copy(r3.at[wid], ri_all)

    def fire(ck, hr, tr, rn, sem):
        pltpu.async_copy(ent_hbm.at[hi_all.at[ck]], hr, sem)
        pltpu.async_copy(ent_hbm.at[ti_all.at[ck]], tr, sem)
        pltpu.async_copy(reln_hbm.at[ri_all.at[ck]], rn, sem)

    def wait3(hr, tr, rn, sem):
        pltpu.make_async_copy(ent_hbm.at[hi_all.at[0]], hr, sem).wait()
        pltpu.make_async_copy(ent_hbm.at[ti_all.at[0]], tr, sem).wait()
        pltpu.make_async_copy(reln_hbm.at[ri_all.at[0]], rn, sem).wait()

    def compute(ck, hrows, trows, rnrows):
        def group_body(g, carry2):
            score_vec = jnp.zeros((LANES,), jnp.float32)
            for k in range(LANES):
                i = g * LANES + k
                d = [hrows[i, pl.ds(j * LANES, LANES)]
                     - trows[i, pl.ds(j * LANES, LANES)]
                     for j in range(SEGS)]
                n = [rnrows[i, pl.ds(EMB_DIM + j * LANES, LANES)]
                     for j in range(SEGS)]
                dot = d[0] * n[0]
                for j in range(1, SEGS):
                    dot = dot + d[j] * n[j]
                s = jnp.sum(dot)
                acc = jnp.zeros((LANES,), jnp.float32)
                for j in range(SEGS):
                    rv = rnrows[i, pl.ds(j * LANES, LANES)]
                    acc = acc + jnp.abs(d[j] + rv - s * n[j])
                score_vec = jnp.where(lanes == k, jnp.sum(acc), score_vec)
            score_all[pl.ds(ck * CHUNK + g * LANES, LANES)] = score_vec
            return carry2

        lax.fori_loop(0, GROUPS, group_body, 0)

    fire(0, hr0, tr0, rn0, sem0)

    def pair_body(p, carry):
        c0 = 2 * p
        fire(c0 + 1, hr1, tr1, rn1, sem1)
        wait3(hr0, tr0, rn0, sem0)
        compute(c0, hr0, tr0, rn0)

        @pl.when(p < N_CHUNKS // 2 - 1)
        def _():
            fire(c0 + 2, hr0, tr0, rn0, sem0)

        wait3(hr1, tr1, rn1, sem1)
        compute(c0 + 1, hr1, tr1, rn1)
        return carry

    lax.fori_loop(0, N_CHUNKS // 2, pair_body, 0)
    pltpu.sync_copy(score_all, out_hbm.at[pl.ds(base, ROWS_PER_WORKER)])


def kernel(h, t, r, ent_weight, rel_weight, norm_weight):
    reln = jnp.concatenate([rel_weight, norm_weight], axis=1)
    h3 = h.astype(jnp.int32).reshape(N_WORKERS, N_CHUNKS, CHUNK)
    t3 = t.astype(jnp.int32).reshape(N_WORKERS, N_CHUNKS, CHUNK)
    r3 = r.astype(jnp.int32).reshape(N_WORKERS, N_CHUNKS, CHUNK)
    mesh = plsc.VectorSubcoreMesh(core_axis_name="c", subcore_axis_name="s")
    run = pl.kernel(
        _sc_body,
        out_type=jax.ShapeDtypeStruct((BATCH_N,), jnp.float32),
        mesh=mesh,
        compiler_params=pltpu.CompilerParams(needs_layout_passes=False),
        scratch_types=[
            pltpu.VMEM((N_CHUNKS, CHUNK), jnp.int32),
            pltpu.VMEM((N_CHUNKS, CHUNK), jnp.int32),
            pltpu.VMEM((N_CHUNKS, CHUNK), jnp.int32),
            pltpu.VMEM((CHUNK, EMB_DIM), jnp.float32),
            pltpu.VMEM((CHUNK, EMB_DIM), jnp.float32),
            pltpu.VMEM((CHUNK, 2 * EMB_DIM), jnp.float32),
            pltpu.VMEM((CHUNK, EMB_DIM), jnp.float32),
            pltpu.VMEM((CHUNK, EMB_DIM), jnp.float32),
            pltpu.VMEM((CHUNK, 2 * EMB_DIM), jnp.float32),
            pltpu.VMEM((ROWS_PER_WORKER,), jnp.float32),
            pltpu.SemaphoreType.DMA,
            pltpu.SemaphoreType.DMA,
        ],
    )
    return run(h3, t3, r3, ent_weight, reln)


# speedup vs baseline: 3.9779x; 1.0810x over previous
import jax
import jax.numpy as jnp
from jax import lax
from jax.experimental import pallas as pl
from jax.experimental.pallas import tpu as pltpu
from jax.experimental.pallas import tpu_sc as plsc

EMB_DIM = 128
BATCH_N = 16384
N_CORES = 2
N_SUBCORES = 16
LANES = 16
SEGS = EMB_DIM // LANES
N_WORKERS = N_CORES * N_SUBCORES
ROWS_PER_WORKER = BATCH_N // N_WORKERS
CHUNK = 64
N_CHUNKS = ROWS_PER_WORKER // CHUNK
GROUPS = CHUNK // LANES


def _sc_body(h_hbm, t_hbm, r_hbm, ent_hbm, rel_hbm, norm_hbm, out_hbm,
             hi_all, ti_all, ri_all,
             hr0, tr0, rr0, nr0, hr1, tr1, rr1, nr1, score_all, sem0, sem1):
    wid = lax.axis_index("s") * N_CORES + lax.axis_index("c")
    base = wid * ROWS_PER_WORKER
    lanes = lax.iota(jnp.int32, LANES)

    pltpu.sync_copy(h_hbm.at[pl.ds(base, ROWS_PER_WORKER)], hi_all)
    pltpu.sync_copy(t_hbm.at[pl.ds(base, ROWS_PER_WORKER)], ti_all)
    pltpu.sync_copy(r_hbm.at[pl.ds(base, ROWS_PER_WORKER)], ri_all)

    def fire(ck, hr, tr, rr, nr, sem):
        sl = pl.ds(ck * CHUNK, CHUNK)
        pltpu.async_copy(ent_hbm.at[hi_all.at[sl]], hr, sem)
        pltpu.async_copy(ent_hbm.at[ti_all.at[sl]], tr, sem)
        pltpu.async_copy(rel_hbm.at[ri_all.at[sl]], rr, sem)
        pltpu.async_copy(norm_hbm.at[ri_all.at[sl]], nr, sem)

    def wait4(hr, tr, rr, nr, sem):
        sl = pl.ds(0, CHUNK)
        pltpu.make_async_copy(ent_hbm.at[hi_all.at[sl]], hr, sem).wait()
        pltpu.make_async_copy(ent_hbm.at[ti_all.at[sl]], tr, sem).wait()
        pltpu.make_async_copy(rel_hbm.at[ri_all.at[sl]], rr, sem).wait()
        pltpu.make_async_copy(norm_hbm.at[ri_all.at[sl]], nr, sem).wait()

    def compute(ck, hrows, trows, rrows, nrows):
        def group_body(g, carry2):
            score_vec = jnp.zeros((LANES,), jnp.float32)
            for k in range(LANES):
                i = g * LANES + k
                d = [hrows[i, pl.ds(j * LANES, LANES)]
                     - trows[i, pl.ds(j * LANES, LANES)]
                     for j in range(SEGS)]
                n = [nrows[i, pl.ds(j * LANES, LANES)]
                     for j in range(SEGS)]
                dot = d[0] * n[0]
                for j in range(1, SEGS):
                    dot = dot + d[j] * n[j]
                s = jnp.sum(dot)
                acc = jnp.zeros((LANES,), jnp.float32)
                for j in range(SEGS):
                    rv = rrows[i, pl.ds(j * LANES, LANES)]
                    acc = acc + jnp.abs(d[j] + rv - s * n[j])
                score_vec = jnp.where(lanes == k, jnp.sum(acc), score_vec)
            score_all[pl.ds(ck * CHUNK + g * LANES, LANES)] = score_vec
            return carry2

        lax.fori_loop(0, GROUPS, group_body, 0)

    fire(0, hr0, tr0, rr0, nr0, sem0)

    def pair_body(p, carry):
        c0 = 2 * p
        fire(c0 + 1, hr1, tr1, rr1, nr1, sem1)
        wait4(hr0, tr0, rr0, nr0, sem0)
        compute(c0, hr0, tr0, rr0, nr0)

        @pl.when(p < N_CHUNKS // 2 - 1)
        def _():
            fire(c0 + 2, hr0, tr0, rr0, nr0, sem0)

        wait4(hr1, tr1, rr1, nr1, sem1)
        compute(c0 + 1, hr1, tr1, rr1, nr1)
        return carry

    lax.fori_loop(0, N_CHUNKS // 2, pair_body, 0)
    pltpu.sync_copy(score_all, out_hbm.at[pl.ds(base, ROWS_PER_WORKER)])


def kernel(h, t, r, ent_weight, rel_weight, norm_weight):
    mesh = plsc.VectorSubcoreMesh(core_axis_name="c", subcore_axis_name="s")
    run = pl.kernel(
        _sc_body,
        out_type=jax.ShapeDtypeStruct((BATCH_N,), jnp.float32),
        mesh=mesh,
        compiler_params=pltpu.CompilerParams(needs_layout_passes=False),
        scratch_types=[
            pltpu.VMEM((ROWS_PER_WORKER,), jnp.int32),
            pltpu.VMEM((ROWS_PER_WORKER,), jnp.int32),
            pltpu.VMEM((ROWS_PER_WORKER,), jnp.int32),
            pltpu.VMEM((CHUNK, EMB_DIM), jnp.float32),
            pltpu.VMEM((CHUNK, EMB_DIM), jnp.float32),
            pltpu.VMEM((CHUNK, EMB_DIM), jnp.float32),
            pltpu.VMEM((CHUNK, EMB_DIM), jnp.float32),
            pltpu.VMEM((CHUNK, EMB_DIM), jnp.float32),
            pltpu.VMEM((CHUNK, EMB_DIM), jnp.float32),
            pltpu.VMEM((CHUNK, EMB_DIM), jnp.float32),
            pltpu.VMEM((CHUNK, EMB_DIM), jnp.float32),
            pltpu.VMEM((ROWS_PER_WORKER,), jnp.float32),
            pltpu.SemaphoreType.DMA,
            pltpu.SemaphoreType.DMA,
        ],
    )
    return run(h.astype(jnp.int32), t.astype(jnp.int32), r.astype(jnp.int32),
               ent_weight, rel_weight, norm_weight)
